# Initial kernel scaffold; baseline (speedup 1.0000x reference)
import jax
import jax.numpy as jnp
from jax.experimental import pallas as pl


def kernel(epoch, epochs, drug_feature, drug_adj, ibatch, mutation_data, gexpr_data, copy_number_data, params):
    raise NotImplementedError("write your pallas kernel here")


# speedup vs baseline: 10.8178x; 10.8178x over previous
import functools

import jax
import jax.numpy as jnp
from jax import lax
from jax.experimental import pallas as pl
from jax.experimental.pallas import tpu as pltpu
from jax.experimental.pallas import tpu_sc as plsc

N_NODES = 10000
NPAD = 10240
N_EDGES = 320000
N_GRAPHS = 128
NCORES = 2
NSUB = 16
NTILES = NCORES * NSUB
EPT = N_EDGES // NTILES
K = 80
NBLK = EPT // K
RPT = NPAD // NSUB
ZR = 16


@functools.lru_cache(maxsize=None)
def _make_agg(C, with_gather=True):
    mesh = plsc.VectorSubcoreMesh(core_axis_name="c", subcore_axis_name="s")

    def agg(x_hbm, row_hbm, col_hbm, out_hbm, acc, rv, cv, rows, zb, sem):
        cid = lax.axis_index("c")
        sid = lax.axis_index("s")
        g = cid * NSUB + sid

        zvec = jnp.zeros((16,), jnp.float32)

        def zb_row(i, carry):
            def zb_col(k, c2):
                zb[i, pl.ds(k * 16, 16)] = zvec
                return c2
            return lax.fori_loop(0, C // 16, zb_col, carry)

        lax.fori_loop(0, ZR, zb_row, 0)
        nb = sid * RPT

        def zcp(i, carry):
            pltpu.sync_copy(zb, acc.at[pl.ds(nb + i * ZR, ZR)])
            return carry

        lax.fori_loop(0, RPT // ZR, zcp, 0)

        if not with_gather:
            onev = jnp.ones((16,), jnp.float32)

            def ones_row(i, carry):
                def ones_col(k, c2):
                    rows[i, pl.ds(k * 16, 16)] = onev
                    return c2
                return lax.fori_loop(0, C // 16, ones_col, carry)

            lax.fori_loop(0, K, ones_row, 0)

        if with_gather:
            pltpu.sync_copy(row_hbm.at[g], rv)
        pltpu.sync_copy(col_hbm.at[g], cv)
        plsc.subcore_barrier()

        def step(j, carry):
            if with_gather:
                pltpu.async_copy(x_hbm.at[rv.at[j]], rows, sem).wait()
            pltpu.sync_copy(rows, acc.at[cv.at[j]], add=True)
            return carry

        lax.fori_loop(0, NBLK, step, 0)
        plsc.subcore_barrier()

        pltpu.sync_copy(acc.at[pl.ds(nb, RPT)], out_hbm.at[cid, pl.ds(nb, RPT)])

    return pl.kernel(
        agg,
        mesh=mesh,
        out_type=jax.ShapeDtypeStruct((NCORES, NPAD, C), jnp.float32),
        scratch_types=[
            pltpu.VMEM_SHARED((NPAD, C), jnp.float32),
            pltpu.VMEM((NBLK, K), jnp.int32),
            pltpu.VMEM((NBLK, K), jnp.int32),
            pltpu.VMEM((K, C), jnp.float32),
            pltpu.VMEM((ZR, C), jnp.float32),
            pltpu.SemaphoreType.DMA,
        ],
    )


def _agg_sum(x, row3, col3):
    parts = _make_agg(x.shape[1])(x, row3, col3)
    return (parts[0] + parts[1])[:N_NODES]


def _degrees(row3, col3):
    parts = _make_agg(128, with_gather=False)(
        jnp.zeros((8, 128), jnp.float32), row3, col3)
    return (parts[0, :N_NODES, 0] + parts[1, :N_NODES, 0]) + 1.0


def _bnorm(x, g, b):
    m = jnp.mean(x, axis=0)
    v = jnp.var(x, axis=0)
    return (x - m) / jnp.sqrt(v + 1e-5) * g + b


def kernel(epoch, epochs, drug_feature, drug_adj, ibatch, mutation_data,
           gexpr_data, copy_number_data, params):
    p = params
    row3 = drug_adj[0].reshape(NTILES, NBLK, K)
    col3 = drug_adj[1].reshape(NTILES, NBLK, K)

    deg = _degrees(row3, col3)
    dinv = lax.rsqrt(deg)

    def sg_agg(x):
        w = x.shape[1]
        if w % 128:
            x = jnp.pad(x, ((0, 0), (0, 128 - w % 128)))
        xp = dinv[:, None] * x
        chunks = []
        for c0 in range(0, xp.shape[1], 128):
            chunks.append(_agg_sum(xp[:, c0:c0 + 128], row3, col3))
        s = jnp.concatenate(chunks, axis=1) if len(chunks) > 1 else chunks[0]
        return (dinv[:, None] * (s + xp))[:, :w]

    x = sg_agg(drug_feature) @ p['W_sg1'].T + p['b_sg1']
    x = _bnorm(jax.nn.relu(x), p['bn1_g'], p['bn1_b'])
    x = sg_agg(x) @ p['W_g0'].T + p['b_g0']
    x = _bnorm(jax.nn.relu(x), p['bng0_g'], p['bng0_b'])
    x = sg_agg(x) @ p['W_g1'].T + p['b_g1']
    x = _bnorm(jax.nn.relu(x), p['bng1_g'], p['bng1_b'])
    x = sg_agg(x @ p['W_end'].T) + p['b_end']
    x = _bnorm(jax.nn.relu(x), p['bne_g'], p['bne_b'])
    x_drug = jax.ops.segment_max(x, ibatch, num_segments=N_GRAPHS)

    h = mutation_data
    h = lax.conv_general_dilated(h, p['Wc1'], window_strides=(1, 5),
                                 padding='VALID',
                                 dimension_numbers=('NCHW', 'OIHW', 'NCHW'))
    h = jnp.tanh(h + p['bc1'][None, :, None, None])
    h = lax.reduce_window(h, -jnp.inf, lax.max, (1, 1, 1, 5), (1, 1, 1, 5), 'VALID')
    h = lax.conv_general_dilated(h, p['Wc2'], window_strides=(1, 2),
                                 padding='VALID',
                                 dimension_numbers=('NCHW', 'OIHW', 'NCHW'))
    h = jax.nn.relu(h + p['bc2'][None, :, None, None])
    h = lax.reduce_window(h, -jnp.inf, lax.max, (1, 1, 1, 10), (1, 1, 1, 10), 'VALID')
    h = h.reshape(h.shape[0], -1)
    x_mut = jax.nn.relu(h @ p['Wmut'].T + p['bmut'])

    g = jax.nn.sigmoid(gexpr_data @ p['Wgex1'].T + p['bgex1'])
    g = _bnorm(g, p['bngex_g'], p['bngex_b'])
    g = jax.nn.relu(g @ p['Wgex2'].T + p['bgex2'])

    c = jnp.tanh(copy_number_data @ p['Wmet1'].T + p['bmet1'])
    c = _bnorm(c, p['bnmet_g'], p['bnmet_b'])
    c = jax.nn.relu(c @ p['Wmet2'].T + p['bmet2'])

    x_cell = jnp.concatenate([x_mut, g, c], axis=1)
    x_cell = jax.nn.leaky_relu(x_cell @ p['Wcat'].T + p['bcat'],
                               negative_slope=0.01)
    x_all = jnp.concatenate([x_cell, x_drug], axis=0)
    return _bnorm(x_all, p['bnc_g'], p['bnc_b'])
